# baseline (speedup 1.0000x reference)
import functools

import jax
import jax.numpy as jnp
from jax import lax
from jax.experimental import pallas as pl
from jax.experimental.pallas import tpu as pltpu
from jax.experimental.pallas import tpu_sc as plsc

_NC = 2
_NS = 16
_L = 16
_NW = _NC * _NS


def _make_sc_kernel(B, D, NU, NH):
    bpw = B // _NW
    mesh = plsc.VectorSubcoreMesh(core_axis_name="c", subcore_axis_name="s")

    @functools.partial(
        pl.kernel,
        mesh=mesh,
        compiler_params=pltpu.CompilerParams(
            needs_layout_passes=False, use_tc_tiling_on_sc=False),
        out_type=jax.ShapeDtypeStruct((B,), jnp.float32),
        scratch_types=[
            pltpu.VMEM((bpw,), jnp.int32),
            pltpu.VMEM((bpw,), jnp.int32),
            pltpu.VMEM((bpw, D), jnp.float32),
            pltpu.VMEM((bpw, D), jnp.float32),
            pltpu.VMEM((bpw,), jnp.float32),
            pltpu.VMEM((bpw,), jnp.float32),
            pltpu.VMEM((bpw,), jnp.float32),
            pltpu.VMEM_SHARED((1, NU), jnp.float32),
            pltpu.VMEM_SHARED((1, NH), jnp.float32),
            pltpu.SemaphoreType.DMA,
            pltpu.SemaphoreType.DMA,
        ],
    )
    def k(uid_hbm, hid_hbm, uemb_hbm, hemb_hbm, ubT_hbm, hbT_hbm, out_hbm,
          idx_u, idx_h, urows, hrows, ub_v, hb_v, out_v, ub_sp, hb_sp,
          sem, sem2):
        wid = lax.axis_index("s") * _NC + lax.axis_index("c")
        sid = lax.axis_index("s")
        base = wid * bpw
        pltpu.sync_copy(uid_hbm.at[pl.ds(base, bpw)], idx_u)
        pltpu.sync_copy(hid_hbm.at[pl.ds(base, bpw)], idx_h)
        cu = pltpu.async_copy(uemb_hbm.at[idx_u], urows, sem)
        ch = pltpu.async_copy(hemb_hbm.at[idx_h], hrows, sem)

        @pl.when(sid == 0)
        def _():
            pltpu.sync_copy(ubT_hbm, ub_sp)
            pltpu.sync_copy(hbT_hbm, hb_sp)

        plsc.subcore_barrier()
        cub = pltpu.async_copy(ub_sp.at[0].at[idx_u], ub_v, sem2)
        chb = pltpu.async_copy(hb_sp.at[0].at[idx_h], hb_v, sem2)
        cu.wait()
        ch.wait()
        cub.wait()
        chb.wait()

        lane = lax.iota(jnp.int32, _L)

        def body(g, carry):
            ids = g * _L + lane
            sl = pl.ds(g * _L, _L)
            acc = ub_v[sl] + hb_v[sl]
            for d in range(D):
                dcol = jnp.full((_L,), d, jnp.int32)
                acc = acc + plsc.load_gather(urows, [ids, dcol]) * plsc.load_gather(
                    hrows, [ids, dcol])
            out_v[sl] = acc
            return carry

        lax.fori_loop(0, bpw // _L, body, 0)
        pltpu.sync_copy(out_v, out_hbm.at[pl.ds(base, bpw)])

    return k


def kernel(user_id_input, hotel_id_input, user_embeddings, hotel_embeddings,
           user_biases, hotel_biases):
    B = user_id_input.shape[0]
    D = user_embeddings.shape[1]
    NU = user_biases.shape[0]
    NH = hotel_biases.shape[0]
    k = _make_sc_kernel(B, D, NU, NH)
    return k(user_id_input.astype(jnp.int32), hotel_id_input.astype(jnp.int32),
             user_embeddings, hotel_embeddings,
             user_biases.T, hotel_biases.T)
